# Initial kernel scaffold; baseline (speedup 1.0000x reference)
import jax
import jax.numpy as jnp
from jax.experimental import pallas as pl


def kernel(Z, send, recv, dist, emb, e_w1, e_b1, e_w2, e_b2, c_w0, c_b0, c_w1, c_b1, c_w2, c_b2, r_w, r_b):
    raise NotImplementedError("write your pallas kernel here")


# speedup vs baseline: 1.8387x; 1.8387x over previous
import functools

import jax
import jax.numpy as jnp
from jax import lax
from jax.experimental import pallas as pl
from jax.experimental.pallas import tpu as pltpu
from jax.experimental.pallas import tpu_sc as plsc

_N = 10000
_E = 160000
_NODE = 256
_EDGE = 256
_NRBF = 32
_RCUT = 5.0

_F32 = jnp.float32

_TE = 1600
_NC = 2
_NS = 16
_EPT = _E // _NS
_K = 80
_NCH = _EPT // _K
_RPT = 640
_RPT_LAST = _N - _RPT * (_NS - 1)
_TN = 2000


def _edge_mlp_body(d_ref, w1_ref, b1_ref, w2_ref, b2_ref, o0_ref, o1_ref):
    d = d_ref[...]
    centers = lax.broadcasted_iota(jnp.int32, (_TE, _NRBF), 1).astype(_F32) * (
        _RCUT / (_NRBF - 1))
    rbf = jnp.exp(-((d - centers) ** 2) * 5.0)
    h = jnp.dot(rbf, w1_ref[...], preferred_element_type=_F32) + b1_ref[...]
    h = h * jax.nn.sigmoid(h)
    e = jnp.dot(h, w2_ref[...], preferred_element_type=_F32) + b2_ref[...]
    e = e * jax.nn.sigmoid(e)
    o0_ref[...] = e[:, :128]
    o1_ref[...] = e[:, 128:]


def _edge_mlp(dist2, e_w1, e_b1, e_w2, e_b2):
    n_blocks = _E // _TE
    return pl.pallas_call(
        _edge_mlp_body,
        grid=(n_blocks,),
        in_specs=[
            pl.BlockSpec((_TE, 1), lambda i: (i, 0)),
            pl.BlockSpec((_NRBF, _EDGE), lambda i: (0, 0)),
            pl.BlockSpec((1, _EDGE), lambda i: (0, 0)),
            pl.BlockSpec((_EDGE, _EDGE), lambda i: (0, 0)),
            pl.BlockSpec((1, _EDGE), lambda i: (0, 0)),
        ],
        out_specs=[
            pl.BlockSpec((_TE, 128), lambda i: (i, 0)),
            pl.BlockSpec((_TE, 128), lambda i: (i, 0)),
        ],
        out_shape=[
            jax.ShapeDtypeStruct((_E, 128), _F32),
            jax.ShapeDtypeStruct((_E, 128), _F32),
        ],
    )(dist2, e_w1, e_b1, e_w2, e_b2)


def _scatter_body(e0_hbm, e1_hbm, recv_hbm, zrows_hbm, m0_hbm, m1_hbm,
                  ed_v, idx_v, acc_sh):
    c = lax.axis_index("c")
    s = lax.axis_index("s")

    rbase = s * _RPT

    @pl.when(s < _NS - 1)
    def _():
        pltpu.sync_copy(zrows_hbm, acc_sh.at[pl.ds(rbase, _RPT), :])

    @pl.when(s == _NS - 1)
    def _():
        pltpu.sync_copy(zrows_hbm.at[pl.ds(0, _RPT_LAST), :],
                        acc_sh.at[pl.ds(rbase, _RPT_LAST), :])

    plsc.subcore_barrier()

    ebase = s * _EPT

    def body(j, carry):
        off = ebase + j * _K
        pltpu.sync_copy(recv_hbm.at[pl.ds(off, _K)], idx_v)

        @pl.when(c == 0)
        def _():
            pltpu.sync_copy(e0_hbm.at[pl.ds(off, _K), :], ed_v)

        @pl.when(c == 1)
        def _():
            pltpu.sync_copy(e1_hbm.at[pl.ds(off, _K), :], ed_v)

        pltpu.sync_copy(ed_v, acc_sh.at[idx_v], add=True)
        return carry

    lax.fori_loop(0, _NCH, body, 0)
    plsc.subcore_barrier()

    @pl.when(jnp.logical_and(c == 0, s < _NS - 1))
    def _():
        pltpu.sync_copy(acc_sh.at[pl.ds(rbase, _RPT), :],
                        m0_hbm.at[pl.ds(rbase, _RPT), :])

    @pl.when(jnp.logical_and(c == 0, s == _NS - 1))
    def _():
        pltpu.sync_copy(acc_sh.at[pl.ds(rbase, _RPT_LAST), :],
                        m0_hbm.at[pl.ds(rbase, _RPT_LAST), :])

    @pl.when(jnp.logical_and(c == 1, s < _NS - 1))
    def _():
        pltpu.sync_copy(acc_sh.at[pl.ds(rbase, _RPT), :],
                        m1_hbm.at[pl.ds(rbase, _RPT), :])

    @pl.when(jnp.logical_and(c == 1, s == _NS - 1))
    def _():
        pltpu.sync_copy(acc_sh.at[pl.ds(rbase, _RPT_LAST), :],
                        m1_hbm.at[pl.ds(rbase, _RPT_LAST), :])


def _scatter(e0, e1, recv, zrows):
    mesh = plsc.VectorSubcoreMesh(
        core_axis_name="c", subcore_axis_name="s",
        num_cores=_NC, num_subcores=_NS)
    return pl.kernel(
        _scatter_body,
        out_type=(
            jax.ShapeDtypeStruct((_N, 128), _F32),
            jax.ShapeDtypeStruct((_N, 128), _F32),
        ),
        mesh=mesh,
        scratch_types=[
            pltpu.VMEM((_K, 128), _F32),
            pltpu.VMEM((_K,), jnp.int32),
            pltpu.VMEM_SHARED((_N, 128), _F32),
        ],
    )(e0, e1, recv, zrows)


def _node_mlp_body(z_ref, m0_ref, m1_ref, emb_ref,
                   w0_ref, b0_ref, w1_ref, b1_ref, w2_ref, b2_ref,
                   rw_ref, rb_ref, out_ref):
    i = pl.program_id(0)

    zi = z_ref[...]
    onehot = (lax.broadcasted_iota(jnp.int32, (_TN, 128), 1) == zi).astype(_F32)
    x = jnp.dot(onehot, emb_ref[...], preferred_element_type=_F32)

    m0 = m0_ref[...]
    m1 = m1_ref[...]
    for w_ref, b_ref in ((w0_ref, b0_ref), (w1_ref, b1_ref), (w2_ref, b2_ref)):
        acc = jnp.dot(x, w_ref[0:_NODE, :], preferred_element_type=_F32)
        acc += jnp.dot(m0, w_ref[_NODE:_NODE + 128, :],
                       preferred_element_type=_F32)
        acc += jnp.dot(m1, w_ref[_NODE + 128:_NODE + 256, :],
                       preferred_element_type=_F32)
        acc += b_ref[...]
        x = acc * jax.nn.sigmoid(acc)

    h = jnp.dot(x, rw_ref[...], preferred_element_type=_F32) + rb_ref[...]
    h = h * jax.nn.sigmoid(h)
    part = jnp.sum(h, axis=0, keepdims=True) * (1.0 / _N)

    @pl.when(i == 0)
    def _():
        out_ref[...] = jnp.zeros_like(out_ref)

    out_ref[...] += part


def _node_mlp(z2, m0, m1, emb_pad, c_w0, c_b0, c_w1, c_b1, c_w2, c_b2,
              r_w, r_b):
    n_blocks = _N // _TN
    full = lambda shape: pl.BlockSpec(shape, lambda i: tuple(0 for _ in shape))
    return pl.pallas_call(
        _node_mlp_body,
        grid=(n_blocks,),
        in_specs=[
            pl.BlockSpec((_TN, 1), lambda i: (i, 0)),
            pl.BlockSpec((_TN, 128), lambda i: (i, 0)),
            pl.BlockSpec((_TN, 128), lambda i: (i, 0)),
            full((128, _NODE)),
            full((_NODE + _EDGE, _NODE)), full((1, _NODE)),
            full((_NODE + _EDGE, _NODE)), full((1, _NODE)),
            full((_NODE + _EDGE, _NODE)), full((1, _NODE)),
            full((_NODE, 256)), full((1, 256)),
        ],
        out_specs=pl.BlockSpec((1, 256), lambda i: (0, 0)),
        out_shape=jax.ShapeDtypeStruct((1, 256), _F32),
    )(z2, m0, m1, emb_pad, c_w0, c_b0, c_w1, c_b1, c_w2, c_b2, r_w, r_b)


@jax.jit
def kernel(Z, send, recv, dist, emb, e_w1, e_b1, e_w2, e_b2,
           c_w0, c_b0, c_w1, c_b1, c_w2, c_b2, r_w, r_b):
    del send

    dist2 = dist.reshape(_E, 1)
    e0, e1 = _edge_mlp(dist2, e_w1, e_b1.reshape(1, _EDGE),
                       e_w2, e_b2.reshape(1, _EDGE))

    zrows = jnp.zeros((_RPT, 128), _F32)
    m0, m1 = _scatter(e0, e1, recv.astype(jnp.int32), zrows)

    emb_pad = jnp.zeros((128, _NODE), _F32).at[:101, :].set(emb)
    z2 = Z.astype(jnp.int32).reshape(_N, 1)
    g = _node_mlp(z2, m0, m1, emb_pad,
                  c_w0, c_b0.reshape(1, _NODE),
                  c_w1, c_b1.reshape(1, _NODE),
                  c_w2, c_b2.reshape(1, _NODE),
                  r_w, r_b.reshape(1, 256))
    return g.reshape(256)


# speedup vs baseline: 2.3012x; 1.2515x over previous
import functools

import jax
import jax.numpy as jnp
from jax import lax
from jax.experimental import pallas as pl
from jax.experimental.pallas import tpu as pltpu
from jax.experimental.pallas import tpu_sc as plsc

_N = 10000
_E = 160000
_NODE = 256
_EDGE = 256
_NRBF = 32
_RCUT = 5.0

_F32 = jnp.float32
_BF16 = jnp.bfloat16

_TE = 1600
_NC = 2
_NS = 16
_EPT = _E // _NS
_K = 80
_NCH = _EPT // _K
_NBUF = 2
_RPT = 640
_RPT_LAST = _N - _RPT * (_NS - 1)
_TN = 2000


def _edge_mlp_body(d_ref, w1_ref, b1_ref, w2_ref, b2_ref, o0_ref, o1_ref):
    d = d_ref[...]
    centers = lax.broadcasted_iota(jnp.int32, (_TE, _NRBF), 1).astype(_F32) * (
        _RCUT / (_NRBF - 1))
    rbf = jnp.exp(-((d - centers) ** 2) * 5.0).astype(_BF16)
    h = jnp.dot(rbf, w1_ref[...], preferred_element_type=_F32) + b1_ref[...]
    h = h * jax.nn.sigmoid(h)
    e = jnp.dot(h.astype(_BF16), w2_ref[...],
                preferred_element_type=_F32) + b2_ref[...]
    e = e * jax.nn.sigmoid(e)
    o0_ref[...] = e[:, :128]
    o1_ref[...] = e[:, 128:]


def _edge_mlp(dist2, e_w1, e_b1, e_w2, e_b2):
    n_blocks = _E // _TE
    return pl.pallas_call(
        _edge_mlp_body,
        grid=(n_blocks,),
        in_specs=[
            pl.BlockSpec((_TE, 1), lambda i: (i, 0)),
            pl.BlockSpec((_NRBF, _EDGE), lambda i: (0, 0)),
            pl.BlockSpec((1, _EDGE), lambda i: (0, 0)),
            pl.BlockSpec((_EDGE, _EDGE), lambda i: (0, 0)),
            pl.BlockSpec((1, _EDGE), lambda i: (0, 0)),
        ],
        out_specs=[
            pl.BlockSpec((_TE, 128), lambda i: (i, 0)),
            pl.BlockSpec((_TE, 128), lambda i: (i, 0)),
        ],
        out_shape=[
            jax.ShapeDtypeStruct((_E, 128), _F32),
            jax.ShapeDtypeStruct((_E, 128), _F32),
        ],
    )(dist2, e_w1, e_b1, e_w2, e_b2)


def _scatter_body(e0_hbm, e1_hbm, recv_hbm, zrows_hbm, m0_hbm, m1_hbm,
                  ed0_v, ed1_v, idx0_v, idx1_v,
                  acc_sh, sem0, sem1, ssem0, ssem1):
    c = lax.axis_index("c")
    s = lax.axis_index("s")

    rbase = s * _RPT

    @pl.when(s < _NS - 1)
    def _():
        pltpu.sync_copy(zrows_hbm, acc_sh.at[pl.ds(rbase, _RPT), :])

    @pl.when(s == _NS - 1)
    def _():
        pltpu.sync_copy(zrows_hbm.at[pl.ds(0, _RPT_LAST), :],
                        acc_sh.at[pl.ds(rbase, _RPT_LAST), :])

    plsc.subcore_barrier()

    ebase = s * _EPT
    bufs = (ed0_v, ed1_v)
    idxs = (idx0_v, idx1_v)
    sems = (sem0, sem1)
    ssems = (ssem0, ssem1)

    def start(j, b):
        off = ebase + j * _K
        pltpu.async_copy(recv_hbm.at[pl.ds(off, _K)], idxs[b], sems[b])

        @pl.when(c == 0)
        def _():
            pltpu.async_copy(e0_hbm.at[pl.ds(off, _K), :], bufs[b], sems[b])

        @pl.when(c == 1)
        def _():
            pltpu.async_copy(e1_hbm.at[pl.ds(off, _K), :], bufs[b], sems[b])

    def wait_load(b):
        pltpu.make_async_copy(
            recv_hbm.at[pl.ds(0, _K)], idxs[b], sems[b]).wait()
        pltpu.make_async_copy(
            e0_hbm.at[pl.ds(0, _K), :], bufs[b], sems[b]).wait()

    def wait_scatter(b):
        pltpu.make_async_copy(bufs[b], acc_sh.at[idxs[b]], ssems[b]).wait()

    start(0, 0)

    def body(g, carry):
        for b in range(_NBUF):
            j = g * _NBUF + b
            wait_load(b)
            pltpu.async_copy(bufs[b], acc_sh.at[idxs[b]], ssems[b], add=True)

            @pl.when(j + 1 < _NCH)
            def _():
                @pl.when(j >= 1)
                def _():
                    wait_scatter(1 - b)

                start(j + 1, 1 - b)
        return carry

    lax.fori_loop(0, _NCH // _NBUF, body, 0)
    wait_load(0)
    pltpu.async_copy(bufs[0], acc_sh.at[idxs[0]], ssems[0], add=True)
    wait_scatter(1)
    wait_scatter(0)
    plsc.subcore_barrier()

    @pl.when(jnp.logical_and(c == 0, s < _NS - 1))
    def _():
        pltpu.sync_copy(acc_sh.at[pl.ds(rbase, _RPT), :],
                        m0_hbm.at[pl.ds(rbase, _RPT), :])

    @pl.when(jnp.logical_and(c == 0, s == _NS - 1))
    def _():
        pltpu.sync_copy(acc_sh.at[pl.ds(rbase, _RPT_LAST), :],
                        m0_hbm.at[pl.ds(rbase, _RPT_LAST), :])

    @pl.when(jnp.logical_and(c == 1, s < _NS - 1))
    def _():
        pltpu.sync_copy(acc_sh.at[pl.ds(rbase, _RPT), :],
                        m1_hbm.at[pl.ds(rbase, _RPT), :])

    @pl.when(jnp.logical_and(c == 1, s == _NS - 1))
    def _():
        pltpu.sync_copy(acc_sh.at[pl.ds(rbase, _RPT_LAST), :],
                        m1_hbm.at[pl.ds(rbase, _RPT_LAST), :])


def _scatter(e0, e1, recv, zrows):
    mesh = plsc.VectorSubcoreMesh(
        core_axis_name="c", subcore_axis_name="s",
        num_cores=_NC, num_subcores=_NS)
    return pl.kernel(
        _scatter_body,
        out_type=(
            jax.ShapeDtypeStruct((_N, 128), _F32),
            jax.ShapeDtypeStruct((_N, 128), _F32),
        ),
        mesh=mesh,
        scratch_types=[
            pltpu.VMEM((_K, 128), _F32),
            pltpu.VMEM((_K, 128), _F32),
            pltpu.VMEM((_K,), jnp.int32),
            pltpu.VMEM((_K,), jnp.int32),
            pltpu.VMEM_SHARED((_N, 128), _F32),
            pltpu.SemaphoreType.DMA,
            pltpu.SemaphoreType.DMA,
            pltpu.SemaphoreType.DMA,
            pltpu.SemaphoreType.DMA,
        ],
    )(e0, e1, recv, zrows)


def _node_mlp_body(z_ref, m0_ref, m1_ref, emb_ref,
                   w0_ref, b0_ref, w1_ref, b1_ref, w2_ref, b2_ref,
                   rw_ref, rb_ref, out_ref):
    i = pl.program_id(0)

    zi = z_ref[...]
    onehot = (lax.broadcasted_iota(jnp.int32, (_TN, 128), 1) == zi).astype(
        _BF16)
    x = jnp.dot(onehot, emb_ref[...], preferred_element_type=_F32)

    m0 = m0_ref[...].astype(_BF16)
    m1 = m1_ref[...].astype(_BF16)
    for w_ref, b_ref in ((w0_ref, b0_ref), (w1_ref, b1_ref), (w2_ref, b2_ref)):
        xb = x.astype(_BF16)
        acc = jnp.dot(xb, w_ref[0:_NODE, :], preferred_element_type=_F32)
        acc += jnp.dot(m0, w_ref[_NODE:_NODE + 128, :],
                       preferred_element_type=_F32)
        acc += jnp.dot(m1, w_ref[_NODE + 128:_NODE + 256, :],
                       preferred_element_type=_F32)
        acc += b_ref[...]
        x = acc * jax.nn.sigmoid(acc)

    h = jnp.dot(x.astype(_BF16), rw_ref[...],
                preferred_element_type=_F32) + rb_ref[...]
    h = h * jax.nn.sigmoid(h)
    part = jnp.sum(h, axis=0, keepdims=True) * (1.0 / _N)

    @pl.when(i == 0)
    def _():
        out_ref[...] = jnp.zeros_like(out_ref)

    out_ref[...] += part


def _node_mlp(z2, m0, m1, emb_pad, c_w0, c_b0, c_w1, c_b1, c_w2, c_b2,
              r_w, r_b):
    n_blocks = _N // _TN
    full = lambda shape: pl.BlockSpec(shape, lambda i: tuple(0 for _ in shape))
    return pl.pallas_call(
        _node_mlp_body,
        grid=(n_blocks,),
        in_specs=[
            pl.BlockSpec((_TN, 1), lambda i: (i, 0)),
            pl.BlockSpec((_TN, 128), lambda i: (i, 0)),
            pl.BlockSpec((_TN, 128), lambda i: (i, 0)),
            full((128, _NODE)),
            full((_NODE + _EDGE, _NODE)), full((1, _NODE)),
            full((_NODE + _EDGE, _NODE)), full((1, _NODE)),
            full((_NODE + _EDGE, _NODE)), full((1, _NODE)),
            full((_NODE, 256)), full((1, 256)),
        ],
        out_specs=pl.BlockSpec((1, 256), lambda i: (0, 0)),
        out_shape=jax.ShapeDtypeStruct((1, 256), _F32),
    )(z2, m0, m1, emb_pad, c_w0, c_b0, c_w1, c_b1, c_w2, c_b2, r_w, r_b)


@jax.jit
def kernel(Z, send, recv, dist, emb, e_w1, e_b1, e_w2, e_b2,
           c_w0, c_b0, c_w1, c_b1, c_w2, c_b2, r_w, r_b):
    del send

    dist2 = dist.reshape(_E, 1)
    e0, e1 = _edge_mlp(dist2, e_w1.astype(_BF16), e_b1.reshape(1, _EDGE),
                       e_w2.astype(_BF16), e_b2.reshape(1, _EDGE))

    zrows = jnp.zeros((_RPT, 128), _F32)
    m0, m1 = _scatter(e0, e1, recv.astype(jnp.int32), zrows)

    emb_pad = jnp.zeros((128, _NODE), _BF16).at[:101, :].set(
        emb.astype(_BF16))
    z2 = Z.astype(jnp.int32).reshape(_N, 1)
    g = _node_mlp(z2, m0, m1, emb_pad,
                  c_w0.astype(_BF16), c_b0.reshape(1, _NODE),
                  c_w1.astype(_BF16), c_b1.reshape(1, _NODE),
                  c_w2.astype(_BF16), c_b2.reshape(1, _NODE),
                  r_w.astype(_BF16), r_b.reshape(1, 256))
    return g.reshape(256)


# speedup vs baseline: 2.5247x; 1.0971x over previous
import functools

import jax
import jax.numpy as jnp
from jax import lax
from jax.experimental import pallas as pl
from jax.experimental.pallas import tpu as pltpu
from jax.experimental.pallas import tpu_sc as plsc

_N = 10000
_E = 160000
_NODE = 256
_EDGE = 256
_NRBF = 32
_RCUT = 5.0

_F32 = jnp.float32
_BF16 = jnp.bfloat16

_TE = 4000
_NC = 2
_NS = 16
_EPT = _E // _NS
_K = 80
_NCH = _EPT // _K
_NBUF = 2
_RPT = 640
_RPT_LAST = _N - _RPT * (_NS - 1)
_TN = 2000


def _edge_mlp_body(d_ref, w1_ref, b1_ref, w2_ref, b2_ref, o0_ref, o1_ref):
    d = d_ref[...]
    centers = lax.broadcasted_iota(jnp.int32, (_TE, _NRBF), 1).astype(_F32) * (
        _RCUT / (_NRBF - 1))
    rbf = jnp.exp(-((d - centers) ** 2) * 5.0).astype(_BF16)
    h = jnp.dot(rbf, w1_ref[...], preferred_element_type=_F32) + b1_ref[...]
    h = h * jax.nn.sigmoid(h)
    e = jnp.dot(h.astype(_BF16), w2_ref[...],
                preferred_element_type=_F32) + b2_ref[...]
    e = e * jax.nn.sigmoid(e)
    o0_ref[...] = e[:, :128]
    o1_ref[...] = e[:, 128:]


def _edge_mlp(dist2, e_w1, e_b1, e_w2, e_b2):
    n_blocks = _E // _TE
    return pl.pallas_call(
        _edge_mlp_body,
        grid=(n_blocks,),
        in_specs=[
            pl.BlockSpec((_TE, 1), lambda i: (i, 0)),
            pl.BlockSpec((_NRBF, _EDGE), lambda i: (0, 0)),
            pl.BlockSpec((1, _EDGE), lambda i: (0, 0)),
            pl.BlockSpec((_EDGE, _EDGE), lambda i: (0, 0)),
            pl.BlockSpec((1, _EDGE), lambda i: (0, 0)),
        ],
        out_specs=[
            pl.BlockSpec((_TE, 128), lambda i: (i, 0)),
            pl.BlockSpec((_TE, 128), lambda i: (i, 0)),
        ],
        out_shape=[
            jax.ShapeDtypeStruct((_E, 128), _F32),
            jax.ShapeDtypeStruct((_E, 128), _F32),
        ],
    )(dist2, e_w1, e_b1, e_w2, e_b2)


def _scatter_body(e0_hbm, e1_hbm, recv_hbm, zrows_hbm, m0_hbm, m1_hbm,
                  ed0_v, ed1_v, idx0_v, idx1_v,
                  acc_sh, sem0, sem1, ssem0, ssem1):
    c = lax.axis_index("c")
    s = lax.axis_index("s")

    rbase = s * _RPT

    @pl.when(s < _NS - 1)
    def _():
        pltpu.sync_copy(zrows_hbm, acc_sh.at[pl.ds(rbase, _RPT), :])

    @pl.when(s == _NS - 1)
    def _():
        pltpu.sync_copy(zrows_hbm.at[pl.ds(0, _RPT_LAST), :],
                        acc_sh.at[pl.ds(rbase, _RPT_LAST), :])

    plsc.subcore_barrier()

    ebase = s * _EPT
    bufs = (ed0_v, ed1_v)
    idxs = (idx0_v, idx1_v)
    sems = (sem0, sem1)
    ssems = (ssem0, ssem1)

    def start(j, b):
        off = ebase + j * _K
        pltpu.async_copy(recv_hbm.at[pl.ds(off, _K)], idxs[b], sems[b])

        @pl.when(c == 0)
        def _():
            pltpu.async_copy(e0_hbm.at[pl.ds(off, _K), :], bufs[b], sems[b])

        @pl.when(c == 1)
        def _():
            pltpu.async_copy(e1_hbm.at[pl.ds(off, _K), :], bufs[b], sems[b])

    def wait_load(b):
        pltpu.make_async_copy(
            recv_hbm.at[pl.ds(0, _K)], idxs[b], sems[b]).wait()
        pltpu.make_async_copy(
            e0_hbm.at[pl.ds(0, _K), :], bufs[b], sems[b]).wait()

    def wait_scatter(b):
        pltpu.make_async_copy(bufs[b], acc_sh.at[idxs[b]], ssems[b]).wait()

    start(0, 0)

    def body(g, carry):
        for b in range(_NBUF):
            j = g * _NBUF + b
            wait_load(b)
            pltpu.async_copy(bufs[b], acc_sh.at[idxs[b]], ssems[b], add=True)

            @pl.when(j + 1 < _NCH)
            def _():
                @pl.when(j >= 1)
                def _():
                    wait_scatter(1 - b)

                start(j + 1, 1 - b)
        return carry

    lax.fori_loop(0, _NCH // _NBUF, body, 0)
    wait_load(0)
    pltpu.async_copy(bufs[0], acc_sh.at[idxs[0]], ssems[0], add=True)
    wait_scatter(1)
    wait_scatter(0)
    plsc.subcore_barrier()

    @pl.when(jnp.logical_and(c == 0, s < _NS - 1))
    def _():
        pltpu.sync_copy(acc_sh.at[pl.ds(rbase, _RPT), :],
                        m0_hbm.at[pl.ds(rbase, _RPT), :])

    @pl.when(jnp.logical_and(c == 0, s == _NS - 1))
    def _():
        pltpu.sync_copy(acc_sh.at[pl.ds(rbase, _RPT_LAST), :],
                        m0_hbm.at[pl.ds(rbase, _RPT_LAST), :])

    @pl.when(jnp.logical_and(c == 1, s < _NS - 1))
    def _():
        pltpu.sync_copy(acc_sh.at[pl.ds(rbase, _RPT), :],
                        m1_hbm.at[pl.ds(rbase, _RPT), :])

    @pl.when(jnp.logical_and(c == 1, s == _NS - 1))
    def _():
        pltpu.sync_copy(acc_sh.at[pl.ds(rbase, _RPT_LAST), :],
                        m1_hbm.at[pl.ds(rbase, _RPT_LAST), :])


def _scatter(e0, e1, recv, zrows):
    mesh = plsc.VectorSubcoreMesh(
        core_axis_name="c", subcore_axis_name="s",
        num_cores=_NC, num_subcores=_NS)
    return pl.kernel(
        _scatter_body,
        out_type=(
            jax.ShapeDtypeStruct((_N, 128), _F32),
            jax.ShapeDtypeStruct((_N, 128), _F32),
        ),
        mesh=mesh,
        scratch_types=[
            pltpu.VMEM((_K, 128), _F32),
            pltpu.VMEM((_K, 128), _F32),
            pltpu.VMEM((_K,), jnp.int32),
            pltpu.VMEM((_K,), jnp.int32),
            pltpu.VMEM_SHARED((_N, 128), _F32),
            pltpu.SemaphoreType.DMA,
            pltpu.SemaphoreType.DMA,
            pltpu.SemaphoreType.DMA,
            pltpu.SemaphoreType.DMA,
        ],
    )(e0, e1, recv, zrows)


def _node_mlp_body(z_ref, m0_ref, m1_ref, emb_ref,
                   w0_ref, b0_ref, w1_ref, b1_ref, w2_ref, b2_ref,
                   rw_ref, rb_ref, out_ref):
    i = pl.program_id(0)

    zi = z_ref[...]
    onehot = (lax.broadcasted_iota(jnp.int32, (_TN, 128), 1) == zi).astype(
        _BF16)
    x = jnp.dot(onehot, emb_ref[...], preferred_element_type=_F32)

    m0 = m0_ref[...].astype(_BF16)
    m1 = m1_ref[...].astype(_BF16)
    for w_ref, b_ref in ((w0_ref, b0_ref), (w1_ref, b1_ref), (w2_ref, b2_ref)):
        xb = x.astype(_BF16)
        acc = jnp.dot(xb, w_ref[0:_NODE, :], preferred_element_type=_F32)
        acc += jnp.dot(m0, w_ref[_NODE:_NODE + 128, :],
                       preferred_element_type=_F32)
        acc += jnp.dot(m1, w_ref[_NODE + 128:_NODE + 256, :],
                       preferred_element_type=_F32)
        acc += b_ref[...]
        x = acc * jax.nn.sigmoid(acc)

    h = jnp.dot(x.astype(_BF16), rw_ref[...],
                preferred_element_type=_F32) + rb_ref[...]
    h = h * jax.nn.sigmoid(h)
    part = jnp.sum(h, axis=0, keepdims=True) * (1.0 / _N)

    @pl.when(i == 0)
    def _():
        out_ref[...] = jnp.zeros_like(out_ref)

    out_ref[...] += part


def _node_mlp(z2, m0, m1, emb_pad, c_w0, c_b0, c_w1, c_b1, c_w2, c_b2,
              r_w, r_b):
    n_blocks = _N // _TN
    full = lambda shape: pl.BlockSpec(shape, lambda i: tuple(0 for _ in shape))
    return pl.pallas_call(
        _node_mlp_body,
        grid=(n_blocks,),
        in_specs=[
            pl.BlockSpec((_TN, 1), lambda i: (i, 0)),
            pl.BlockSpec((_TN, 128), lambda i: (i, 0)),
            pl.BlockSpec((_TN, 128), lambda i: (i, 0)),
            full((128, _NODE)),
            full((_NODE + _EDGE, _NODE)), full((1, _NODE)),
            full((_NODE + _EDGE, _NODE)), full((1, _NODE)),
            full((_NODE + _EDGE, _NODE)), full((1, _NODE)),
            full((_NODE, 256)), full((1, 256)),
        ],
        out_specs=pl.BlockSpec((1, 256), lambda i: (0, 0)),
        out_shape=jax.ShapeDtypeStruct((1, 256), _F32),
    )(z2, m0, m1, emb_pad, c_w0, c_b0, c_w1, c_b1, c_w2, c_b2, r_w, r_b)


@jax.jit
def kernel(Z, send, recv, dist, emb, e_w1, e_b1, e_w2, e_b2,
           c_w0, c_b0, c_w1, c_b1, c_w2, c_b2, r_w, r_b):
    del send

    dist2 = dist.reshape(_E, 1)
    e0, e1 = _edge_mlp(dist2, e_w1.astype(_BF16), e_b1.reshape(1, _EDGE),
                       e_w2.astype(_BF16), e_b2.reshape(1, _EDGE))

    zrows = jnp.zeros((_RPT, 128), _F32)
    m0, m1 = _scatter(e0, e1, recv.astype(jnp.int32), zrows)

    emb_pad = jnp.zeros((128, _NODE), _BF16).at[:101, :].set(
        emb.astype(_BF16))
    z2 = Z.astype(jnp.int32).reshape(_N, 1)
    g = _node_mlp(z2, m0, m1, emb_pad,
                  c_w0.astype(_BF16), c_b0.reshape(1, _NODE),
                  c_w1.astype(_BF16), c_b1.reshape(1, _NODE),
                  c_w2.astype(_BF16), c_b2.reshape(1, _NODE),
                  r_w.astype(_BF16), r_b.reshape(1, 256))
    return g.reshape(256)


# speedup vs baseline: 2.5774x; 1.0209x over previous
import functools

import jax
import jax.numpy as jnp
from jax import lax
from jax.experimental import pallas as pl
from jax.experimental.pallas import tpu as pltpu
from jax.experimental.pallas import tpu_sc as plsc

_N = 10000
_E = 160000
_NODE = 256
_EDGE = 256
_NRBF = 32
_RCUT = 5.0

_F32 = jnp.float32
_BF16 = jnp.bfloat16

_TE = 8000
_NC = 2
_NS = 16
_EPT = _E // _NS
_K = 80
_NCH = _EPT // _K
_NBUF = 2
_RPT = 640
_RPT_LAST = _N - _RPT * (_NS - 1)
_TN = 5000


def _edge_mlp_body(d_ref, w1_ref, b1_ref, w2_ref, b2_ref, o0_ref, o1_ref):
    d = d_ref[...]
    centers = lax.broadcasted_iota(jnp.int32, (_TE, _NRBF), 1).astype(_F32) * (
        _RCUT / (_NRBF - 1))
    rbf = jnp.exp(-((d - centers) ** 2) * 5.0).astype(_BF16)
    h = jnp.dot(rbf, w1_ref[...], preferred_element_type=_F32) + b1_ref[...]
    h = h * jax.nn.sigmoid(h)
    e = jnp.dot(h.astype(_BF16), w2_ref[...],
                preferred_element_type=_F32) + b2_ref[...]
    e = e * jax.nn.sigmoid(e)
    o0_ref[...] = e[:, :128]
    o1_ref[...] = e[:, 128:]


def _edge_mlp(dist2, e_w1, e_b1, e_w2, e_b2):
    n_blocks = _E // _TE
    return pl.pallas_call(
        _edge_mlp_body,
        grid=(n_blocks,),
        in_specs=[
            pl.BlockSpec((_TE, 1), lambda i: (i, 0)),
            pl.BlockSpec((_NRBF, _EDGE), lambda i: (0, 0)),
            pl.BlockSpec((1, _EDGE), lambda i: (0, 0)),
            pl.BlockSpec((_EDGE, _EDGE), lambda i: (0, 0)),
            pl.BlockSpec((1, _EDGE), lambda i: (0, 0)),
        ],
        out_specs=[
            pl.BlockSpec((_TE, 128), lambda i: (i, 0)),
            pl.BlockSpec((_TE, 128), lambda i: (i, 0)),
        ],
        out_shape=[
            jax.ShapeDtypeStruct((_E, 128), _F32),
            jax.ShapeDtypeStruct((_E, 128), _F32),
        ],
    )(dist2, e_w1, e_b1, e_w2, e_b2)


def _scatter_body(e0_hbm, e1_hbm, recv_hbm, zrows_hbm, m0_hbm, m1_hbm,
                  ed0_v, ed1_v, idx0_v, idx1_v,
                  acc_sh, sem0, sem1, ssem0, ssem1):
    c = lax.axis_index("c")
    s = lax.axis_index("s")

    rbase = s * _RPT

    @pl.when(s < _NS - 1)
    def _():
        pltpu.sync_copy(zrows_hbm, acc_sh.at[pl.ds(rbase, _RPT), :])

    @pl.when(s == _NS - 1)
    def _():
        pltpu.sync_copy(zrows_hbm.at[pl.ds(0, _RPT_LAST), :],
                        acc_sh.at[pl.ds(rbase, _RPT_LAST), :])

    plsc.subcore_barrier()

    ebase = s * _EPT
    bufs = (ed0_v, ed1_v)
    idxs = (idx0_v, idx1_v)
    sems = (sem0, sem1)
    ssems = (ssem0, ssem1)

    def start(j, b):
        off = ebase + j * _K
        pltpu.async_copy(recv_hbm.at[pl.ds(off, _K)], idxs[b], sems[b])

        @pl.when(c == 0)
        def _():
            pltpu.async_copy(e0_hbm.at[pl.ds(off, _K), :], bufs[b], sems[b])

        @pl.when(c == 1)
        def _():
            pltpu.async_copy(e1_hbm.at[pl.ds(off, _K), :], bufs[b], sems[b])

    def wait_load(b):
        pltpu.make_async_copy(
            recv_hbm.at[pl.ds(0, _K)], idxs[b], sems[b]).wait()
        pltpu.make_async_copy(
            e0_hbm.at[pl.ds(0, _K), :], bufs[b], sems[b]).wait()

    def wait_scatter(b):
        pltpu.make_async_copy(bufs[b], acc_sh.at[idxs[b]], ssems[b]).wait()

    start(0, 0)

    def body(g, carry):
        for b in range(_NBUF):
            j = g * _NBUF + b
            wait_load(b)
            pltpu.async_copy(bufs[b], acc_sh.at[idxs[b]], ssems[b], add=True)

            @pl.when(j + 1 < _NCH)
            def _():
                @pl.when(j >= 1)
                def _():
                    wait_scatter(1 - b)

                start(j + 1, 1 - b)
        return carry

    lax.fori_loop(0, _NCH // _NBUF, body, 0)
    wait_load(0)
    pltpu.async_copy(bufs[0], acc_sh.at[idxs[0]], ssems[0], add=True)
    wait_scatter(1)
    wait_scatter(0)
    plsc.subcore_barrier()

    @pl.when(jnp.logical_and(c == 0, s < _NS - 1))
    def _():
        pltpu.sync_copy(acc_sh.at[pl.ds(rbase, _RPT), :],
                        m0_hbm.at[pl.ds(rbase, _RPT), :])

    @pl.when(jnp.logical_and(c == 0, s == _NS - 1))
    def _():
        pltpu.sync_copy(acc_sh.at[pl.ds(rbase, _RPT_LAST), :],
                        m0_hbm.at[pl.ds(rbase, _RPT_LAST), :])

    @pl.when(jnp.logical_and(c == 1, s < _NS - 1))
    def _():
        pltpu.sync_copy(acc_sh.at[pl.ds(rbase, _RPT), :],
                        m1_hbm.at[pl.ds(rbase, _RPT), :])

    @pl.when(jnp.logical_and(c == 1, s == _NS - 1))
    def _():
        pltpu.sync_copy(acc_sh.at[pl.ds(rbase, _RPT_LAST), :],
                        m1_hbm.at[pl.ds(rbase, _RPT_LAST), :])


def _scatter(e0, e1, recv, zrows):
    mesh = plsc.VectorSubcoreMesh(
        core_axis_name="c", subcore_axis_name="s",
        num_cores=_NC, num_subcores=_NS)
    return pl.kernel(
        _scatter_body,
        out_type=(
            jax.ShapeDtypeStruct((_N, 128), _F32),
            jax.ShapeDtypeStruct((_N, 128), _F32),
        ),
        mesh=mesh,
        scratch_types=[
            pltpu.VMEM((_K, 128), _F32),
            pltpu.VMEM((_K, 128), _F32),
            pltpu.VMEM((_K,), jnp.int32),
            pltpu.VMEM((_K,), jnp.int32),
            pltpu.VMEM_SHARED((_N, 128), _F32),
            pltpu.SemaphoreType.DMA,
            pltpu.SemaphoreType.DMA,
            pltpu.SemaphoreType.DMA,
            pltpu.SemaphoreType.DMA,
        ],
    )(e0, e1, recv, zrows)


def _node_mlp_body(z_ref, m0_ref, m1_ref, emb_ref,
                   w0_ref, b0_ref, w1_ref, b1_ref, w2_ref, b2_ref,
                   rw_ref, rb_ref, out_ref):
    i = pl.program_id(0)

    zi = z_ref[...]
    onehot = (lax.broadcasted_iota(jnp.int32, (_TN, 128), 1) == zi).astype(
        _BF16)
    x = jnp.dot(onehot, emb_ref[...], preferred_element_type=_F32)

    m0 = m0_ref[...].astype(_BF16)
    m1 = m1_ref[...].astype(_BF16)
    for w_ref, b_ref in ((w0_ref, b0_ref), (w1_ref, b1_ref), (w2_ref, b2_ref)):
        xb = x.astype(_BF16)
        acc = jnp.dot(xb, w_ref[0:_NODE, :], preferred_element_type=_F32)
        acc += jnp.dot(m0, w_ref[_NODE:_NODE + 128, :],
                       preferred_element_type=_F32)
        acc += jnp.dot(m1, w_ref[_NODE + 128:_NODE + 256, :],
                       preferred_element_type=_F32)
        acc += b_ref[...]
        x = acc * jax.nn.sigmoid(acc)

    h = jnp.dot(x.astype(_BF16), rw_ref[...],
                preferred_element_type=_F32) + rb_ref[...]
    h = h * jax.nn.sigmoid(h)
    part = jnp.sum(h, axis=0, keepdims=True) * (1.0 / _N)

    @pl.when(i == 0)
    def _():
        out_ref[...] = jnp.zeros_like(out_ref)

    out_ref[...] += part


def _node_mlp(z2, m0, m1, emb_pad, c_w0, c_b0, c_w1, c_b1, c_w2, c_b2,
              r_w, r_b):
    n_blocks = _N // _TN
    full = lambda shape: pl.BlockSpec(shape, lambda i: tuple(0 for _ in shape))
    return pl.pallas_call(
        _node_mlp_body,
        grid=(n_blocks,),
        in_specs=[
            pl.BlockSpec((_TN, 1), lambda i: (i, 0)),
            pl.BlockSpec((_TN, 128), lambda i: (i, 0)),
            pl.BlockSpec((_TN, 128), lambda i: (i, 0)),
            full((128, _NODE)),
            full((_NODE + _EDGE, _NODE)), full((1, _NODE)),
            full((_NODE + _EDGE, _NODE)), full((1, _NODE)),
            full((_NODE + _EDGE, _NODE)), full((1, _NODE)),
            full((_NODE, 256)), full((1, 256)),
        ],
        out_specs=pl.BlockSpec((1, 256), lambda i: (0, 0)),
        out_shape=jax.ShapeDtypeStruct((1, 256), _F32),
    )(z2, m0, m1, emb_pad, c_w0, c_b0, c_w1, c_b1, c_w2, c_b2, r_w, r_b)


@jax.jit
def kernel(Z, send, recv, dist, emb, e_w1, e_b1, e_w2, e_b2,
           c_w0, c_b0, c_w1, c_b1, c_w2, c_b2, r_w, r_b):
    del send

    dist2 = dist.reshape(_E, 1)
    e0, e1 = _edge_mlp(dist2, e_w1.astype(_BF16), e_b1.reshape(1, _EDGE),
                       e_w2.astype(_BF16), e_b2.reshape(1, _EDGE))

    zrows = jnp.zeros((_RPT, 128), _F32)
    m0, m1 = _scatter(e0, e1, recv.astype(jnp.int32), zrows)

    emb_pad = jnp.zeros((128, _NODE), _BF16).at[:101, :].set(
        emb.astype(_BF16))
    z2 = Z.astype(jnp.int32).reshape(_N, 1)
    g = _node_mlp(z2, m0, m1, emb_pad,
                  c_w0.astype(_BF16), c_b0.reshape(1, _NODE),
                  c_w1.astype(_BF16), c_b1.reshape(1, _NODE),
                  c_w2.astype(_BF16), c_b2.reshape(1, _NODE),
                  r_w.astype(_BF16), r_b.reshape(1, 256))
    return g.reshape(256)
